# Initial kernel scaffold; baseline (speedup 1.0000x reference)
import jax
import jax.numpy as jnp
from jax.experimental import pallas as pl


def kernel(x_cat, x_num, tables, W1, b1, W2, b2):
    raise NotImplementedError("write your pallas kernel here")


# speedup vs baseline: 4.1022x; 4.1022x over previous
import jax
import jax.numpy as jnp
from jax import lax
from jax.experimental import pallas as pl
from jax.experimental.pallas import tpu as pltpu
from jax.experimental.pallas import tpu_sc as plsc

NUM_FIELDS = 26
VOCAB = 100000
EMB = 50
BATCH = 16384
NUM_FEAT = 13
HIDDEN = 128
OUT = 2

EMB_PAD = 56
EMB_TOT = NUM_FIELDS * EMB_PAD
N_ROWS = BATCH * NUM_FIELDS
V_TOT = NUM_FIELDS * VOCAB

NC = 2
NS = 16
NW = NC * NS
GATHER_W = 128
STEPS_PER_TILE = N_ROWS // (NW * GATHER_W)


def _gather_body(tbl_hbm, idx_hbm, out_hbm, idx_v, rows_v, sem):
    wid = lax.axis_index("s") * NC + lax.axis_index("c")
    base = wid * STEPS_PER_TILE * GATHER_W

    def step(i, carry):
        off = base + i * GATHER_W
        pltpu.sync_copy(idx_hbm.at[pl.ds(off, GATHER_W)], idx_v)
        pltpu.async_copy(tbl_hbm.at[idx_v], rows_v, sem).wait()
        pltpu.sync_copy(rows_v, out_hbm.at[pl.ds(off, GATHER_W)])
        return carry

    lax.fori_loop(0, STEPS_PER_TILE, step, 0)


def _sc_gather(tables_pad, idx_flat):
    mesh = plsc.VectorSubcoreMesh(core_axis_name="c", subcore_axis_name="s")
    return pl.kernel(
        _gather_body,
        out_type=jax.ShapeDtypeStruct((N_ROWS, EMB_PAD), jnp.float32),
        mesh=mesh,
        scratch_types=[
            pltpu.VMEM((GATHER_W,), jnp.int32),
            pltpu.VMEM((GATHER_W, EMB_PAD), jnp.float32),
            pltpu.SemaphoreType.DMA,
        ],
        compiler_params=pltpu.CompilerParams(use_tc_tiling_on_sc=False),
    )(tables_pad, idx_flat)


def _mlp_body(emb_ref, xnum_ref, w1e_ref, w1n_ref, b1_ref, w2_ref, b2_ref,
              out_ref):
    h = jnp.dot(emb_ref[...], w1e_ref[...],
                preferred_element_type=jnp.float32)
    h = h + jnp.dot(xnum_ref[...], w1n_ref[...],
                    preferred_element_type=jnp.float32)
    h = jnp.maximum(h + b1_ref[...], 0.0)
    out_ref[...] = jnp.dot(h, w2_ref[...],
                           preferred_element_type=jnp.float32) + b2_ref[...]


def _tc_mlp(emb, x_num, W1e, W1n, b1, W2, b2, block_b):
    grid = (BATCH // block_b,)
    return pl.pallas_call(
        _mlp_body,
        grid=grid,
        in_specs=[
            pl.BlockSpec((block_b, EMB_TOT), lambda i: (i, 0)),
            pl.BlockSpec((block_b, NUM_FEAT), lambda i: (i, 0)),
            pl.BlockSpec((EMB_TOT, HIDDEN), lambda i: (0, 0)),
            pl.BlockSpec((NUM_FEAT, HIDDEN), lambda i: (0, 0)),
            pl.BlockSpec((1, HIDDEN), lambda i: (0, 0)),
            pl.BlockSpec((HIDDEN, OUT), lambda i: (0, 0)),
            pl.BlockSpec((1, OUT), lambda i: (0, 0)),
        ],
        out_specs=pl.BlockSpec((block_b, OUT), lambda i: (i, 0)),
        out_shape=jax.ShapeDtypeStruct((BATCH, OUT), jnp.float32),
    )(emb, x_num, W1e, W1n, b1, W2, b2)


def kernel(x_cat, x_num, tables, W1, b1, W2, b2):
    idx_flat = (x_cat.astype(jnp.int32)
                + (jnp.arange(NUM_FIELDS, dtype=jnp.int32) * VOCAB)[None, :]
                ).reshape(-1)
    tables_pad = jnp.pad(tables, ((0, 0), (0, 0), (0, EMB_PAD - EMB))
                         ).reshape(V_TOT, EMB_PAD)
    rows = _sc_gather(tables_pad, idx_flat)
    emb = rows.reshape(BATCH, EMB_TOT)
    W1e_pad = (jnp.zeros((NUM_FIELDS, EMB_PAD, HIDDEN), dtype=W1.dtype)
               .at[:, :EMB, :]
               .set(W1[:NUM_FIELDS * EMB].reshape(NUM_FIELDS, EMB, HIDDEN))
               ).reshape(EMB_TOT, HIDDEN)
    out = _tc_mlp(emb, x_num,
                  W1e_pad, W1[NUM_FIELDS * EMB:],
                  b1.reshape(1, HIDDEN), W2, b2.reshape(1, OUT),
                  block_b=2048)
    return out


# speedup vs baseline: 4.5971x; 1.1206x over previous
import jax
import jax.numpy as jnp
from jax import lax
from jax.experimental import pallas as pl
from jax.experimental.pallas import tpu as pltpu
from jax.experimental.pallas import tpu_sc as plsc

NUM_FIELDS = 26
VOCAB = 100000
EMB = 50
BATCH = 16384
NUM_FEAT = 13
HIDDEN = 128
OUT = 2

EMB_PAD = 56
EMB_TOT = NUM_FIELDS * EMB_PAD
N_ROWS = BATCH * NUM_FIELDS
V_TOT = NUM_FIELDS * VOCAB

NC = 2
NS = 16
NW = NC * NS
GATHER_W = 128
STEPS_PER_TILE = N_ROWS // (NW * GATHER_W)


def _gather_body(tbl_hbm, idx_hbm, out_hbm, idx_v, rows_v, sem):
    wid = lax.axis_index("s") * NC + lax.axis_index("c")
    base = wid * STEPS_PER_TILE * GATHER_W

    def step(i, carry):
        off = base + i * GATHER_W
        pltpu.sync_copy(idx_hbm.at[pl.ds(off, GATHER_W)], idx_v)
        pltpu.async_copy(tbl_hbm.at[idx_v], rows_v, sem).wait()
        pltpu.sync_copy(rows_v, out_hbm.at[pl.ds(off, GATHER_W)])
        return carry

    lax.fori_loop(0, STEPS_PER_TILE, step, 0)


def _sc_gather(tables_pad, idx_flat):
    mesh = plsc.VectorSubcoreMesh(core_axis_name="c", subcore_axis_name="s")
    return pl.kernel(
        _gather_body,
        out_type=jax.ShapeDtypeStruct((N_ROWS, EMB_PAD), jnp.float32),
        mesh=mesh,
        scratch_types=[
            pltpu.VMEM((GATHER_W,), jnp.int32),
            pltpu.VMEM((GATHER_W, EMB_PAD), jnp.float32),
            pltpu.SemaphoreType.DMA,
        ],
        compiler_params=pltpu.CompilerParams(use_tc_tiling_on_sc=False),
    )(tables_pad, idx_flat)


def _mlp_body(emb_ref, xnum_ref, w1e_ref, w1n_ref, b1_ref, w2_ref, b2_ref,
              out_ref):
    h = jnp.dot(emb_ref[...], w1e_ref[...],
                preferred_element_type=jnp.float32)
    h = h + jnp.dot(xnum_ref[...], w1n_ref[...],
                    preferred_element_type=jnp.float32)
    h = jnp.maximum(h + b1_ref[...], 0.0)
    out_ref[...] = jnp.dot(h, w2_ref[...],
                           preferred_element_type=jnp.float32) + b2_ref[...]


def _tc_mlp(emb, x_num, W1e, W1n, b1, W2, b2, block_b):
    grid = (BATCH // block_b,)
    return pl.pallas_call(
        _mlp_body,
        grid=grid,
        in_specs=[
            pl.BlockSpec((block_b, EMB_TOT), lambda i: (i, 0)),
            pl.BlockSpec((block_b, NUM_FEAT), lambda i: (i, 0)),
            pl.BlockSpec((EMB_TOT, HIDDEN), lambda i: (0, 0)),
            pl.BlockSpec((NUM_FEAT, HIDDEN), lambda i: (0, 0)),
            pl.BlockSpec((1, HIDDEN), lambda i: (0, 0)),
            pl.BlockSpec((HIDDEN, OUT), lambda i: (0, 0)),
            pl.BlockSpec((1, OUT), lambda i: (0, 0)),
        ],
        out_specs=pl.BlockSpec((block_b, OUT), lambda i: (i, 0)),
        out_shape=jax.ShapeDtypeStruct((BATCH, OUT), jnp.float32),
    )(emb, x_num, W1e, W1n, b1, W2, b2)


def kernel(x_cat, x_num, tables, W1, b1, W2, b2):
    idx_flat = (x_cat.astype(jnp.int32)
                + (jnp.arange(NUM_FIELDS, dtype=jnp.int32) * VOCAB)[None, :]
                ).reshape(-1)
    pad_proj = jnp.eye(EMB, EMB_PAD, dtype=jnp.float32)
    tables_pad = (tables @ pad_proj).reshape(V_TOT, EMB_PAD)
    rows = _sc_gather(tables_pad, idx_flat)
    emb = rows.reshape(BATCH, EMB_TOT)
    W1e_pad = (jnp.zeros((NUM_FIELDS, EMB_PAD, HIDDEN), dtype=W1.dtype)
               .at[:, :EMB, :]
               .set(W1[:NUM_FIELDS * EMB].reshape(NUM_FIELDS, EMB, HIDDEN))
               ).reshape(EMB_TOT, HIDDEN)
    out = _tc_mlp(emb, x_num,
                  W1e_pad, W1[NUM_FIELDS * EMB:],
                  b1.reshape(1, HIDDEN), W2, b2.reshape(1, OUT),
                  block_b=2048)
    return out


# speedup vs baseline: 19.3270x; 4.2042x over previous
import jax
import jax.numpy as jnp
from jax import lax
from jax.experimental import pallas as pl
from jax.experimental.pallas import tpu as pltpu
from jax.experimental.pallas import tpu_sc as plsc

NUM_FIELDS = 26
VOCAB = 100000
EMB = 50
BATCH = 16384
NUM_FEAT = 13
HIDDEN = 128
OUT = 2

V_TOT = NUM_FIELDS * VOCAB

NC = 2
NS = 16
NW = NC * NS
GATHER_W = 128
B_PER_TILE = BATCH // NW
SUBBLOCKS = B_PER_TILE // GATHER_W
N_TBLK = BATCH // GATHER_W


def _gather_body(proj_hbm, idx_hbm, out_hbm, idx_v, rows_v, sem):
    wid = lax.axis_index("s") * NC + lax.axis_index("c")
    tblk = wid * SUBBLOCKS
    pltpu.sync_copy(idx_hbm.at[pl.ds(tblk, SUBBLOCKS)], idx_v)

    def sub(s, carry):
        pltpu.async_copy(proj_hbm.at[idx_v.at[s, 0]], rows_v, sem).wait()
        copies = [
            pltpu.async_copy(proj_hbm.at[idx_v.at[s, f]], rows_v, sem,
                             add=True)
            for f in range(1, NUM_FIELDS)
        ]
        for c in copies:
            c.wait()
        pltpu.sync_copy(rows_v,
                        out_hbm.at[pl.ds((tblk + s) * GATHER_W, GATHER_W)])
        return carry

    lax.fori_loop(0, SUBBLOCKS, sub, 0)


def _sc_gather_add(proj, idx3):
    mesh = plsc.VectorSubcoreMesh(core_axis_name="c", subcore_axis_name="s")
    return pl.kernel(
        _gather_body,
        out_type=jax.ShapeDtypeStruct((BATCH, HIDDEN), jnp.float32),
        mesh=mesh,
        scratch_types=[
            pltpu.VMEM((SUBBLOCKS, NUM_FIELDS, GATHER_W), jnp.int32),
            pltpu.VMEM((GATHER_W, HIDDEN), jnp.float32),
            pltpu.SemaphoreType.DMA,
        ],
        compiler_params=pltpu.CompilerParams(use_tc_tiling_on_sc=False),
    )(proj, idx3)


def _mlp_body(hsum_ref, xnum_ref, w1n_ref, b1_ref, w2_ref, b2_ref, out_ref):
    h = hsum_ref[...] + jnp.dot(xnum_ref[...], w1n_ref[...],
                                preferred_element_type=jnp.float32)
    h = jnp.maximum(h + b1_ref[...], 0.0)
    out_ref[...] = jnp.dot(h, w2_ref[...],
                           preferred_element_type=jnp.float32) + b2_ref[...]


def _tc_mlp(hsum, x_num, W1n, b1, W2, b2, block_b):
    grid = (BATCH // block_b,)
    return pl.pallas_call(
        _mlp_body,
        grid=grid,
        in_specs=[
            pl.BlockSpec((block_b, HIDDEN), lambda i: (i, 0)),
            pl.BlockSpec((block_b, NUM_FEAT), lambda i: (i, 0)),
            pl.BlockSpec((NUM_FEAT, HIDDEN), lambda i: (0, 0)),
            pl.BlockSpec((1, HIDDEN), lambda i: (0, 0)),
            pl.BlockSpec((HIDDEN, OUT), lambda i: (0, 0)),
            pl.BlockSpec((1, OUT), lambda i: (0, 0)),
        ],
        out_specs=pl.BlockSpec((block_b, OUT), lambda i: (i, 0)),
        out_shape=jax.ShapeDtypeStruct((BATCH, OUT), jnp.float32),
    )(hsum, x_num, W1n, b1, W2, b2)


def kernel(x_cat, x_num, tables, W1, b1, W2, b2):
    t_T = jnp.transpose(tables, (0, 2, 1))
    W1_3 = W1[:NUM_FIELDS * EMB].reshape(NUM_FIELDS, EMB, HIDDEN)
    proj = jnp.einsum("fex,feh->fxh", t_T, W1_3,
                      preferred_element_type=jnp.float32)
    proj = proj.reshape(V_TOT, HIDDEN)

    idx = (x_cat.astype(jnp.int32)
           + (jnp.arange(NUM_FIELDS, dtype=jnp.int32) * VOCAB)[None, :])
    idx3 = jnp.transpose(idx.reshape(N_TBLK, GATHER_W, NUM_FIELDS),
                         (0, 2, 1))

    hsum = _sc_gather_add(proj, idx3)
    return _tc_mlp(hsum, x_num, W1[NUM_FIELDS * EMB:],
                   b1.reshape(1, HIDDEN), W2, b2.reshape(1, OUT),
                   block_b=2048)
